# Initial kernel scaffold; baseline (speedup 1.0000x reference)
import jax
import jax.numpy as jnp
from jax.experimental import pallas as pl


def kernel(hidden_states, image_mask, audio_mask, Wg, expert_bias, w1, w3, w2, sw1, sw3, sw2):
    raise NotImplementedError("write your pallas kernel here")


# speedup vs baseline: 3.7505x; 3.7505x over previous
import functools
import jax
import jax.numpy as jnp
from jax import lax
from jax.experimental import pallas as pl
from jax.experimental.pallas import tpu as pltpu

T = 2048
H = 768
E = 64
K = 8
G = 8
TG = 4
I = 256
SI = 256
RSF = 2.5
BM = 128
NBLK = (T * K + E * BM) // BM
NEG = -1e30
BIGF = 1e9


def _sigmoid(x):
    return 1.0 / (1.0 + jnp.exp(-x))


def _gate_kernel(x_ref, wg_ref, bias_ref, comb_ref, d_ref, w_ref, cnt_ref):
    x = x_ref[...]
    logits = lax.dot_general(x, wg_ref[...], (((1,), (1,)), ((), ())),
                             preferred_element_type=jnp.float32)
    scores = _sigmoid(logits)
    sr = scores + bias_ref[...]

    iota64 = lax.broadcasted_iota(jnp.int32, (T, E), 1).astype(jnp.float32)
    iota8 = iota64[:, :G]

    gs_cols = []
    for g in range(G):
        sub = sr[:, g * (E // G):(g + 1) * (E // G)]
        subi = iota64[:, :E // G]
        m1 = jnp.max(sub, axis=1, keepdims=True)
        a1 = jnp.min(jnp.where(sub == m1, subi, BIGF), axis=1, keepdims=True)
        sub2 = jnp.where(subi == a1, NEG, sub)
        m2 = jnp.max(sub2, axis=1, keepdims=True)
        gs_cols.append(m1 + m2)
    gs = jnp.concatenate(gs_cols, axis=1)

    gmask = jnp.zeros((T, G), jnp.float32)
    work = gs
    for _ in range(TG):
        m = jnp.max(work, axis=1, keepdims=True)
        a = jnp.min(jnp.where(work == m, iota8, BIGF), axis=1, keepdims=True)
        pick = (iota8 == a).astype(jnp.float32)
        gmask = gmask + pick
        work = jnp.where(pick > 0, NEG, work)

    score_mask = jnp.concatenate(
        [jnp.broadcast_to(gmask[:, g:g + 1], (T, E // G)) for g in range(G)],
        axis=1)
    masked = jnp.where(score_mask > 0, sr, NEG)

    picks = []
    work2 = masked
    sel = jnp.zeros((T, E), jnp.float32)
    for _ in range(K):
        m = jnp.max(work2, axis=1, keepdims=True)
        a = jnp.min(jnp.where(work2 == m, iota64, BIGF), axis=1, keepdims=True)
        pick = (iota64 == a).astype(jnp.float32)
        picks.append(pick)
        sel = sel + pick
        work2 = jnp.where(pick > 0, NEG, work2)

    ssum = jnp.sum(sel * scores, axis=1, keepdims=True)
    scale = RSF / (ssum + 1e-20)
    comb_ref[...] = sel * scores * scale

    CT = 256
    tri = (lax.broadcasted_iota(jnp.int32, (CT, CT), 0) >=
           lax.broadcasted_iota(jnp.int32, (CT, CT), 1)).astype(jnp.float32)
    off = jnp.zeros((1, E), jnp.float32)
    pos_chunks = []
    for c in range(T // CT):
        seg = sel[c * CT:(c + 1) * CT, :]
        cs = lax.dot_general(tri, seg, (((1,), (0,)), ((), ())),
                             preferred_element_type=jnp.float32) + off
        pos_chunks.append(cs)
        off = cs[CT - 1:CT, :]
    posincl = jnp.concatenate(pos_chunks, axis=0)
    counts = off
    pc = jnp.floor((counts + (BM - 1)) * (1.0 / BM)) * BM
    iu_r = lax.broadcasted_iota(jnp.int32, (E, E), 0)
    iu_c = lax.broadcasted_iota(jnp.int32, (E, E), 1)
    su = (iu_r < iu_c).astype(jnp.float32)
    pexcl = lax.dot_general(pc, su, (((1,), (0,)), ((), ())),
                            preferred_element_type=jnp.float32)
    dmat = pexcl + posincl - 1.0

    w_cols = []
    d_cols = []
    for k in range(K):
        pick = picks[k]
        w_cols.append(jnp.sum(pick * scores, axis=1, keepdims=True))
        d_cols.append(jnp.sum(pick * dmat, axis=1, keepdims=True))
    w_ref[...] = jnp.concatenate(w_cols, axis=1) * scale
    d_ref[...] = jnp.concatenate(d_cols, axis=1).astype(jnp.int32)
    cnt_ref[...] = counts


def _gate(x, Wg, expert_bias):
    return pl.pallas_call(
        _gate_kernel,
        out_shape=(
            jax.ShapeDtypeStruct((T, E), jnp.float32),
            jax.ShapeDtypeStruct((T, K), jnp.int32),
            jax.ShapeDtypeStruct((T, K), jnp.float32),
            jax.ShapeDtypeStruct((1, E), jnp.float32),
        ),
    )(x, Wg, expert_bias.reshape(1, E))


def _dense_kernel(x_ref, comb_ref, w1_ref, w3_ref, w2_ref,
                  sw1_ref, sw3_ref, sw2_ref, out_ref, acc_ref):
    e = pl.program_id(0)
    x = x_ref[...]

    @pl.when(e == 0)
    def _():
        acc_ref[...] = jnp.zeros_like(acc_ref)

    g = lax.dot_general(x, w1_ref[0], (((1,), (1,)), ((), ())),
                        preferred_element_type=jnp.float32)
    u = lax.dot_general(x, w3_ref[0], (((1,), (1,)), ((), ())),
                        preferred_element_type=jnp.float32)
    h = g * _sigmoid(g) * u
    o = lax.dot_general(h, w2_ref[0], (((1,), (1,)), ((), ())),
                        preferred_element_type=jnp.float32)
    colid = lax.broadcasted_iota(jnp.int32, (T, E), 1)
    ce = jnp.sum(jnp.where(colid == e, comb_ref[...], 0.0),
                 axis=1, keepdims=True)
    acc_ref[...] += ce * o

    @pl.when(e == E - 1)
    def _():
        sg = lax.dot_general(x, sw1_ref[...], (((1,), (1,)), ((), ())),
                             preferred_element_type=jnp.float32)
        su_ = lax.dot_general(x, sw3_ref[...], (((1,), (1,)), ((), ())),
                              preferred_element_type=jnp.float32)
        sh = sg * _sigmoid(sg) * su_
        so = lax.dot_general(sh, sw2_ref[...], (((1,), (1,)), ((), ())),
                             preferred_element_type=jnp.float32)
        out_ref[...] = acc_ref[...] + so


def kernel(hidden_states, image_mask, audio_mask, Wg, expert_bias,
           w1, w3, w2, sw1, sw3, sw2):
    x = hidden_states.reshape(-1, H)
    combine, d_tk, w_tk, counts = _gate(x, Wg, expert_bias)

    out = pl.pallas_call(
        _dense_kernel,
        grid=(E,),
        in_specs=[
            pl.BlockSpec((T, H), lambda e: (0, 0)),
            pl.BlockSpec((T, E), lambda e: (0, 0)),
            pl.BlockSpec((1, I, H), lambda e: (e, 0, 0)),
            pl.BlockSpec((1, I, H), lambda e: (e, 0, 0)),
            pl.BlockSpec((1, H, I), lambda e: (e, 0, 0)),
            pl.BlockSpec((SI, H), lambda e: (0, 0)),
            pl.BlockSpec((SI, H), lambda e: (0, 0)),
            pl.BlockSpec((H, SI), lambda e: (0, 0)),
        ],
        out_specs=pl.BlockSpec((T, H), lambda e: (0, 0)),
        out_shape=jax.ShapeDtypeStruct((T, H), jnp.float32),
        scratch_shapes=[pltpu.VMEM((T, H), jnp.float32)],
        compiler_params=pltpu.CompilerParams(
            dimension_semantics=("arbitrary",)),
    )(x, combine, w1, w3, w2, sw1, sw3, sw2)
    return out
